# baseline (speedup 1.0000x reference)
import jax
import jax.numpy as jnp
from jax import lax
from jax.experimental import pallas as pl
from jax.experimental.pallas import tpu as pltpu

_CHUNK = 32768


def _mm_body(x_ref, w_ref, o_ref):
    o_ref[...] = lax.dot_general(
        w_ref[...], x_ref[...],
        (((0,), (1,)), ((), ())),
        preferred_element_type=jnp.float32,
    )


def kernel(x, W):
    n, k = x.shape
    m = W.shape[1]
    grid = n // _CHUNK
    out_t = pl.pallas_call(
        _mm_body,
        grid=(grid,),
        in_specs=[
            pl.BlockSpec((_CHUNK, k), lambda i: (i, 0)),
            pl.BlockSpec((k, m), lambda i: (0, 0)),
        ],
        out_specs=pl.BlockSpec((m, _CHUNK), lambda i: (0, i)),
        out_shape=jax.ShapeDtypeStruct((m, n), jnp.float32),
        compiler_params=pltpu.CompilerParams(
            dimension_semantics=("arbitrary",),
        ),
    )(x, W)
    return out_t.T


# speedup vs baseline: 1.0443x; 1.0443x over previous
import jax
import jax.numpy as jnp
from jax import lax
from jax.experimental import pallas as pl
from jax.experimental.pallas import tpu as pltpu

_CHUNK = 8192
_NBUF = 4


def _mm_body(x_hbm, w_ref, o_hbm, xbuf, obuf, insem, outsem):
    nch = x_hbm.shape[0] // _CHUNK
    w = w_ref[...]

    def in_copy(c):
        return pltpu.make_async_copy(
            x_hbm.at[pl.ds(c * _CHUNK, _CHUNK), :],
            xbuf.at[c % _NBUF],
            insem.at[c % _NBUF],
        )

    def out_copy(c):
        return pltpu.make_async_copy(
            obuf.at[c % _NBUF],
            o_hbm.at[:, pl.ds(c * _CHUNK, _CHUNK)],
            outsem.at[c % _NBUF],
        )

    for c in range(min(_NBUF, nch)):
        in_copy(c).start()
    for c in range(nch):
        b = c % _NBUF
        in_copy(c).wait()
        if c >= _NBUF:
            out_copy(c - _NBUF).wait()
        obuf[b] = lax.dot_general(
            w, xbuf[b],
            (((0,), (1,)), ((), ())),
            preferred_element_type=jnp.float32,
        )
        out_copy(c).start()
        if c + _NBUF < nch:
            in_copy(c + _NBUF).start()
    for c in range(max(nch - _NBUF, 0), nch):
        out_copy(c).wait()


def kernel(x, W):
    n, k = x.shape
    m = W.shape[1]
    out_t = pl.pallas_call(
        _mm_body,
        in_specs=[
            pl.BlockSpec(memory_space=pl.ANY),
            pl.BlockSpec(memory_space=pltpu.VMEM),
        ],
        out_specs=pl.BlockSpec(memory_space=pl.ANY),
        out_shape=jax.ShapeDtypeStruct((m, n), jnp.float32),
        scratch_shapes=[
            pltpu.VMEM((_NBUF, _CHUNK, k), jnp.float32),
            pltpu.VMEM((_NBUF, m, _CHUNK), jnp.float32),
            pltpu.SemaphoreType.DMA((_NBUF,)),
            pltpu.SemaphoreType.DMA((_NBUF,)),
        ],
    )(x, W)
    return out_t.T
